# Initial kernel scaffold; baseline (speedup 1.0000x reference)
import functools

import jax
import jax.numpy as jnp
from jax import lax
from jax.experimental import pallas as pl
from jax.experimental.pallas import tpu as pltpu
from jax.experimental.pallas import tpu_sc as plsc

B = 128
V = 32768
L = 16
NC = 2
NS = 16
NW = NC * NS
ROWS_PER_W = B // NW
U = 8
CHUNK = U * L
NITER = V // CHUNK


def _row_max(buf):
    init = tuple(jnp.full((L,), -jnp.inf, jnp.float32) for _ in range(U))

    def body(i, ms):
        base = i * CHUNK
        return tuple(
            jnp.maximum(ms[u], buf[pl.ds(base + u * L, L)]) for u in range(U)
        )

    ms = lax.fori_loop(0, NITER, body, init)
    m = ms[0]
    for u in range(1, U):
        m = jnp.maximum(m, ms[u])
    return jnp.max(m)


def _row_sumexp(buf, row_max):
    init = tuple(jnp.zeros((L,), jnp.float32) for _ in range(U))

    def body(i, ss):
        base = i * CHUNK
        return tuple(
            ss[u] + jnp.exp(buf[pl.ds(base + u * L, L)] - row_max)
            for u in range(U)
        )

    ss = lax.fori_loop(0, NITER, body, init)
    s = ss[0]
    for u in range(1, U):
        s = s + ss[u]
    return jnp.sum(s)


@functools.partial(
    pl.kernel,
    mesh=plsc.VectorSubcoreMesh(core_axis_name="c", subcore_axis_name="s"),
    out_type=jax.ShapeDtypeStruct((NW, L), jnp.float32),
    scratch_types=[
        pltpu.VMEM((V,), jnp.float32),
        pltpu.VMEM((V,), jnp.float32),
        pltpu.VMEM((L,), jnp.float32),
        pltpu.SemaphoreType.DMA,
        pltpu.SemaphoreType.DMA,
    ],
)
def _sc_top1(x_hbm, out_hbm, buf0, buf1, out_buf, sem0, sem1):
    cid = lax.axis_index("c")
    sid = lax.axis_index("s")
    wid = sid * NC + cid
    base_row = wid * ROWS_PER_W

    bufs = (buf0, buf1)
    sems = (sem0, sem1)
    copies = [None, None]

    copies[0] = pltpu.async_copy(x_hbm.at[base_row], bufs[0], sems[0])

    acc = jnp.zeros((L,), jnp.float32)
    lane = lax.iota(jnp.int32, L)
    for r in range(ROWS_PER_W):
        cur = r % 2
        nxt = (r + 1) % 2
        if r + 1 < ROWS_PER_W:
            copies[nxt] = pltpu.async_copy(
                x_hbm.at[base_row + r + 1], bufs[nxt], sems[nxt]
            )
        copies[cur].wait()
        row_max = _row_max(bufs[cur])
        sum_exp = _row_sumexp(bufs[cur], row_max)
        acc = jnp.where(lane == r, 1.0 / sum_exp, acc)

    out_buf[...] = acc
    pltpu.sync_copy(out_buf, out_hbm.at[wid])


def kernel(class_t, dom_res):
    padded = _sc_top1(class_t)
    return padded[:, :ROWS_PER_W].reshape(-1)


# speedup vs baseline: 1.5193x; 1.5193x over previous
import functools

import jax
import jax.numpy as jnp
from jax import lax
from jax.experimental import pallas as pl
from jax.experimental.pallas import tpu as pltpu
from jax.experimental.pallas import tpu_sc as plsc

B = 128
V = 32768
L = 16
NC = 2
NS = 16
NW = NC * NS
ROWS_PER_W = B // NW
U = 8
CHUNK = U * L
NITER = V // CHUNK


def _butterfly(v, op):
    lane = lax.iota(jnp.int32, L)
    for k in (8, 4, 2, 1):
        v = op(v, v.at[lane ^ k].get(mode="promise_in_bounds"))
    return v


def _row_max(buf):
    init = tuple(jnp.full((L,), -jnp.inf, jnp.float32) for _ in range(U))

    def body(i, ms):
        base = i * CHUNK
        return tuple(
            jnp.maximum(ms[u], buf[pl.ds(base + u * L, L)]) for u in range(U)
        )

    ms = lax.fori_loop(0, NITER, body, init)
    m = ms[0]
    for u in range(1, U):
        m = jnp.maximum(m, ms[u])
    return _butterfly(m, jnp.maximum)


def _row_sumexp(buf, row_max):
    init = tuple(jnp.zeros((L,), jnp.float32) for _ in range(U))

    def body(i, ss):
        base = i * CHUNK
        return tuple(
            ss[u] + jnp.exp(buf[pl.ds(base + u * L, L)] - row_max)
            for u in range(U)
        )

    ss = lax.fori_loop(0, NITER, body, init)
    s = ss[0]
    for u in range(1, U):
        s = s + ss[u]
    return _butterfly(s, jnp.add)


@functools.partial(
    pl.kernel,
    mesh=plsc.VectorSubcoreMesh(core_axis_name="c", subcore_axis_name="s"),
    out_type=jax.ShapeDtypeStruct((NW, L), jnp.float32),
    scratch_types=[
        pltpu.VMEM((V,), jnp.float32),
        pltpu.VMEM((V,), jnp.float32),
        pltpu.VMEM((L,), jnp.float32),
        pltpu.SemaphoreType.DMA,
        pltpu.SemaphoreType.DMA,
    ],
)
def _sc_top1(x_hbm, out_hbm, buf0, buf1, out_buf, sem0, sem1):
    cid = lax.axis_index("c")
    sid = lax.axis_index("s")
    wid = sid * NC + cid
    base_row = wid * ROWS_PER_W

    bufs = (buf0, buf1)
    sems = (sem0, sem1)
    copies = [None, None]

    copies[0] = pltpu.async_copy(x_hbm.at[base_row], bufs[0], sems[0])

    acc = jnp.zeros((L,), jnp.float32)
    lane = lax.iota(jnp.int32, L)
    for r in range(ROWS_PER_W):
        cur = r % 2
        nxt = (r + 1) % 2
        if r + 1 < ROWS_PER_W:
            copies[nxt] = pltpu.async_copy(
                x_hbm.at[base_row + r + 1], bufs[nxt], sems[nxt]
            )
        copies[cur].wait()
        row_max = _row_max(bufs[cur])
        sum_exp = _row_sumexp(bufs[cur], row_max)
        acc = jnp.where(lane == r, 1.0 / sum_exp, acc)

    out_buf[...] = acc
    pltpu.sync_copy(out_buf, out_hbm.at[wid])


def kernel(class_t, dom_res):
    padded = _sc_top1(class_t)
    return padded[:, :ROWS_PER_W].reshape(-1)


# speedup vs baseline: 1.5218x; 1.0017x over previous
import functools

import jax
import jax.numpy as jnp
from jax import lax
from jax.experimental import pallas as pl
from jax.experimental.pallas import tpu as pltpu
from jax.experimental.pallas import tpu_sc as plsc

B = 128
V = 32768
L = 16
NC = 2
NS = 16
NW = NC * NS
ROWS_PER_W = B // NW
U = 8
CHUNK = U * L
NITER = V // CHUNK


def _butterfly(v, op):
    lane = lax.iota(jnp.int32, L)
    for k in (8, 4, 2, 1):
        v = op(v, v.at[lane ^ k].get(mode="promise_in_bounds"))
    return v


def _row_max(buf):
    init = tuple(jnp.full((L,), -jnp.inf, jnp.float32) for _ in range(U))

    @plsc.parallel_loop(0, V, CHUNK, unroll=2, carry=init)
    def ms(base, ms):
        return tuple(
            jnp.maximum(ms[u], buf[pl.ds(base + u * L, L)]) for u in range(U)
        )
    m = ms[0]
    for u in range(1, U):
        m = jnp.maximum(m, ms[u])
    return _butterfly(m, jnp.maximum)


def _row_sumexp(buf, row_max):
    init = tuple(jnp.zeros((L,), jnp.float32) for _ in range(U))

    @plsc.parallel_loop(0, V, CHUNK, unroll=2, carry=init)
    def ss(base, ss):
        return tuple(
            ss[u] + jnp.exp(buf[pl.ds(base + u * L, L)] - row_max)
            for u in range(U)
        )
    s = ss[0]
    for u in range(1, U):
        s = s + ss[u]
    return _butterfly(s, jnp.add)


@functools.partial(
    pl.kernel,
    mesh=plsc.VectorSubcoreMesh(core_axis_name="c", subcore_axis_name="s"),
    out_type=jax.ShapeDtypeStruct((NW, L), jnp.float32),
    scratch_types=[
        pltpu.VMEM((V,), jnp.float32),
        pltpu.VMEM((V,), jnp.float32),
        pltpu.VMEM((L,), jnp.float32),
        pltpu.SemaphoreType.DMA,
        pltpu.SemaphoreType.DMA,
    ],
)
def _sc_top1(x_hbm, out_hbm, buf0, buf1, out_buf, sem0, sem1):
    cid = lax.axis_index("c")
    sid = lax.axis_index("s")
    wid = sid * NC + cid
    base_row = wid * ROWS_PER_W

    bufs = (buf0, buf1)
    sems = (sem0, sem1)
    copies = [None, None]

    copies[0] = pltpu.async_copy(x_hbm.at[base_row], bufs[0], sems[0])

    acc = jnp.zeros((L,), jnp.float32)
    lane = lax.iota(jnp.int32, L)
    for r in range(ROWS_PER_W):
        cur = r % 2
        nxt = (r + 1) % 2
        if r + 1 < ROWS_PER_W:
            copies[nxt] = pltpu.async_copy(
                x_hbm.at[base_row + r + 1], bufs[nxt], sems[nxt]
            )
        copies[cur].wait()
        row_max = _row_max(bufs[cur])
        sum_exp = _row_sumexp(bufs[cur], row_max)
        acc = jnp.where(lane == r, 1.0 / sum_exp, acc)

    out_buf[...] = acc
    pltpu.sync_copy(out_buf, out_hbm.at[wid])


def kernel(class_t, dom_res):
    padded = _sc_top1(class_t)
    return padded[:, :ROWS_PER_W].reshape(-1)


# speedup vs baseline: 1.5469x; 1.0165x over previous
import functools

import jax
import jax.numpy as jnp
from jax import lax
from jax.experimental import pallas as pl
from jax.experimental.pallas import tpu as pltpu
from jax.experimental.pallas import tpu_sc as plsc

B = 128
V = 32768
L = 16
NC = 2
NS = 16
NW = NC * NS
ROWS_PER_W = B // NW
U = 8
CHUNK = U * L


def _butterfly(v, op):
    lane = lax.iota(jnp.int32, L)
    for k in (8, 4, 2, 1):
        v = op(v, v.at[lane ^ k].get(mode="promise_in_bounds"))
    return v


def _row_max(buf, base):
    init = tuple(jnp.full((L,), -jnp.inf, jnp.float32) for _ in range(U))

    @plsc.parallel_loop(0, V, CHUNK, carry=init)
    def ms(off, ms):
        return tuple(
            jnp.maximum(ms[u], buf[pl.ds(base + off + u * L, L)])
            for u in range(U)
        )

    m = ms[0]
    for u in range(1, U):
        m = jnp.maximum(m, ms[u])
    return _butterfly(m, jnp.maximum)


def _row_sumexp(buf, base, row_max):
    init = tuple(jnp.zeros((L,), jnp.float32) for _ in range(U))

    @plsc.parallel_loop(0, V, CHUNK, carry=init)
    def ss(off, ss):
        return tuple(
            ss[u] + jnp.exp(buf[pl.ds(base + off + u * L, L)] - row_max)
            for u in range(U)
        )

    s = ss[0]
    for u in range(1, U):
        s = s + ss[u]
    return _butterfly(s, jnp.add)


@functools.partial(
    pl.kernel,
    mesh=plsc.VectorSubcoreMesh(core_axis_name="c", subcore_axis_name="s"),
    out_type=jax.ShapeDtypeStruct((NW, L), jnp.float32),
    scratch_types=[
        pltpu.VMEM((2 * V,), jnp.float32),
        pltpu.VMEM((L,), jnp.float32),
        pltpu.SemaphoreType.DMA((2,)),
    ],
)
def _sc_top1(x_hbm, out_hbm, buf, out_buf, sems):
    cid = lax.axis_index("c")
    sid = lax.axis_index("s")
    wid = sid * NC + cid
    base_row = wid * ROWS_PER_W

    pltpu.make_async_copy(
        x_hbm.at[base_row], buf.at[pl.ds(0, V)], sems.at[0]
    ).start()

    lane = lax.iota(jnp.int32, L)

    def row_body(r, acc):
        cur = lax.rem(r, 2)
        nxt = lax.rem(r + 1, 2)

        @pl.when(r + 1 < ROWS_PER_W)
        def _():
            pltpu.make_async_copy(
                x_hbm.at[base_row + r + 1],
                buf.at[pl.ds(nxt * V, V)],
                sems.at[nxt],
            ).start()

        pltpu.make_async_copy(
            x_hbm.at[base_row + r], buf.at[pl.ds(cur * V, V)], sems.at[cur]
        ).wait()

        base = cur * V
        row_max = _row_max(buf, base)
        sum_exp = _row_sumexp(buf, base, row_max)
        return jnp.where(lane == r, 1.0 / sum_exp, acc)

    acc = lax.fori_loop(0, ROWS_PER_W, row_body, jnp.zeros((L,), jnp.float32))

    out_buf[...] = acc
    pltpu.sync_copy(out_buf, out_hbm.at[wid])


def kernel(class_t, dom_res):
    padded = _sc_top1(class_t)
    return padded[:, :ROWS_PER_W].reshape(-1)
